# baseline (speedup 1.0000x reference)
import functools

import jax
import jax.numpy as jnp
from jax import lax
from jax.experimental import pallas as pl
from jax.experimental.pallas import tpu as pltpu
from jax.experimental.pallas import tpu_sc as plsc

N = 100000
E = 1600000
VOCAB = 5000
EMBED = 32
HID = 100
FC = 32
NG = 64

N_PAD = 102400
E_ROWS = 12544
TRASH = N_PAD - 1
NC = 2
NS = 16
BLK = 2048
NBLK = N_PAD // BLK

F32 = jnp.float32
I32 = jnp.int32

_MESH = plsc.VectorSubcoreMesh(
    core_axis_name="c", subcore_axis_name="s", num_cores=NC, num_subcores=NS)

_SLICE = N_PAD // NS


def _deg_body(e0_ref, e1_ref, e2_ref, zeros_ref, deg_ref, idxbA, idxbB,
              ones_v, hist, semA, semB):
    c = lax.axis_index("c")
    s = lax.axis_index("s")
    for b in range(8):
        ones_v[pl.ds(16 * b, 16)] = jnp.ones((16,), F32)
    rows_per_tile = E_ROWS // NS
    npairs = rows_per_tile // 16
    for g, edges_ref in enumerate((e0_ref, e1_ref, e2_ref)):
        pltpu.sync_copy(zeros_ref, hist.at[pl.ds(s * _SLICE, _SLICE)])
        plsc.subcore_barrier()

        def pair(k, carry):
            for half, (idxb, sem) in enumerate(((idxbA, semA), (idxbB, semB))):
                base = s * rows_per_tile + (2 * k + half) * 8

                @pl.when(k > 0)
                def _():
                    pltpu.make_async_copy(edges_ref.at[0, pl.ds(0, 8)],
                                          idxb, sem).wait()

                pltpu.sync_copy(edges_ref.at[c, pl.ds(base, 8)], idxb)
                for b in range(8):
                    pltpu.async_copy(ones_v, hist.at[idxb.at[b]], sem,
                                     add=True)
            return carry

        lax.fori_loop(0, npairs, pair, 0)
        for idxb, sem in ((idxbA, semA), (idxbB, semB)):
            pltpu.make_async_copy(edges_ref.at[0, pl.ds(0, 8)],
                                  idxb, sem).wait()
        plsc.subcore_barrier()
        off = (c * 3 + g) * N_PAD + s * _SLICE
        pltpu.sync_copy(hist.at[pl.ds(s * _SLICE, _SLICE)],
                        deg_ref.at[pl.ds(off, _SLICE)])
        plsc.subcore_barrier()


_k_deg = functools.partial(
    pl.kernel,
    out_type=jax.ShapeDtypeStruct((NC * 3 * N_PAD,), F32),
    mesh=_MESH,
    compiler_params=pltpu.CompilerParams(use_tc_tiling_on_sc=False),
    scratch_types=[
        pltpu.VMEM((8, 128), I32),
        pltpu.VMEM((8, 128), I32),
        pltpu.VMEM((128,), F32),
        pltpu.VMEM_SHARED((N_PAD,), F32),
        pltpu.SemaphoreType.DMA,
        pltpu.SemaphoreType.DMA,
    ],
)(_deg_body)


_EROWS = 2432


def _emb_body(feats_ref, embed_ref, x_ref, idxbA, idxbB, rowsA, rowsB,
              sem_g, semA, semB):
    wid = lax.axis_index("s") * NC + lax.axis_index("c")
    rows_per_w = _EROWS // (NC * NS)
    base0 = wid * rows_per_w

    def do_group(gi, idxb, rows, sem, first):
        base = base0 + gi * 4

        @pl.when(jnp.logical_not(first))
        def _():
            pltpu.make_async_copy(x_ref.at[pl.ds(0, 512)], rows, sem).wait()

        pltpu.sync_copy(feats_ref.at[pl.ds(base, 4)], idxb)
        cps = [pltpu.async_copy(embed_ref.at[idxb.at[b]],
                                rows.at[pl.ds(b * 128, 128)], sem_g)
               for b in range(4)]
        for cp in cps:
            cp.wait()
        for b in range(4):
            pltpu.async_copy(rows.at[pl.ds(b * 128, 128)],
                             x_ref.at[pl.ds((base + b) * 128, 128)], sem)

    def pair(k, carry):
        do_group(2 * k, idxbA, rowsA, semA, k == 0)
        do_group(2 * k + 1, idxbB, rowsB, semB, k == 0)
        return carry

    lax.fori_loop(0, 9, pair, 0)
    do_group(18, idxbA, rowsA, semA, jnp.bool_(False))
    for rows, sem in ((rowsA, semA), (rowsB, semB)):
        pltpu.make_async_copy(x_ref.at[pl.ds(0, 512)], rows, sem).wait()


_k_emb = functools.partial(
    pl.kernel,
    out_type=jax.ShapeDtypeStruct((_EROWS * 128, EMBED), F32),
    mesh=_MESH,
    compiler_params=pltpu.CompilerParams(use_tc_tiling_on_sc=False),
    scratch_types=[
        pltpu.VMEM((4, 128), I32),
        pltpu.VMEM((4, 128), I32),
        pltpu.VMEM((512, EMBED), F32),
        pltpu.VMEM((512, EMBED), F32),
        pltpu.SemaphoreType.DMA,
        pltpu.SemaphoreType.DMA,
        pltpu.SemaphoreType.DMA,
    ],
)(_emb_body)


def _agg_body(e0_ref, e1_ref, e2_ref, xlo_ref, xhi_ref,
              zeros_ref, part_ref,
              idxsA, idxdA, idxsB, idxdB, rowsA, rowsB, agg,
              sem_g, semA, semB, siA, siB):
    c = lax.axis_index("c")
    s = lax.axis_index("s")
    rows_per_sc = E_ROWS // NC
    rows_per_tile = rows_per_sc // NS
    e_tab = (e0_ref, e1_ref, e2_ref)
    for g in range(3):
        edges_ref = e_tab[g]
        for p in range(2):
            xs_ref = (xlo_ref if p == 0 else xhi_ref).at[g]
            pltpu.sync_copy(zeros_ref, agg.at[pl.ds(s * _SLICE, _SLICE)])
            plsc.subcore_barrier()
            base0 = c * rows_per_sc + s * rows_per_tile

            def fire_idx(gi, idx_s, idx_d, sem):
                base = base0 + gi * 4
                pltpu.async_copy(edges_ref.at[0, pl.ds(base, 4)], idx_s, sem)
                pltpu.async_copy(edges_ref.at[1, pl.ds(base, 4)], idx_d, sem)

            def wait_idx(idx_s, idx_d, sem):
                pltpu.make_async_copy(edges_ref.at[0, pl.ds(0, 4)],
                                      idx_s, sem).wait()
                pltpu.make_async_copy(edges_ref.at[1, pl.ds(0, 4)],
                                      idx_d, sem).wait()

            def drain_sc(rows, sem):
                pltpu.make_async_copy(zeros_ref.at[pl.ds(0, 512)],
                                      rows, sem).wait()

            def gather_scatter(idx_s, idx_d, rows, sem):
                cps = [pltpu.async_copy(xs_ref.at[idx_s.at[b]],
                                        rows.at[pl.ds(b * 128, 128)], sem_g)
                       for b in range(4)]
                for cp in cps:
                    cp.wait()
                for b in range(4):
                    pltpu.async_copy(rows.at[pl.ds(b * 128, 128)],
                                     agg.at[idx_d.at[b]], sem, add=True)

            fire_idx(0, idxsA, idxdA, siA)

            def pair(k, carry):
                @pl.when(k > 0)
                def _():
                    drain_sc(rowsB, semB)
                fire_idx(2 * k + 1, idxsB, idxdB, siB)
                wait_idx(idxsA, idxdA, siA)
                gather_scatter(idxsA, idxdA, rowsA, semA)
                wait_idx(idxsB, idxdB, siB)
                gather_scatter(idxsB, idxdB, rowsB, semB)

                @pl.when(k < 48)
                def _():
                    drain_sc(rowsA, semA)
                    fire_idx(2 * k + 2, idxsA, idxdA, siA)
                return carry

            lax.fori_loop(0, 49, pair, 0)
            drain_sc(rowsA, semA)
            drain_sc(rowsB, semB)
            plsc.subcore_barrier()
            pltpu.sync_copy(agg.at[pl.ds(s * _SLICE, _SLICE)],
                            part_ref.at[g * 4 + p * NC + c,
                                        pl.ds(s * _SLICE, _SLICE)])
            plsc.subcore_barrier()


_k_agg = functools.partial(
    pl.kernel,
    out_type=jax.ShapeDtypeStruct((3 * 2 * NC, N_PAD, 16), F32),
    mesh=_MESH,
    compiler_params=pltpu.CompilerParams(use_tc_tiling_on_sc=False),
    scratch_types=[
        pltpu.VMEM((4, 128), I32),
        pltpu.VMEM((4, 128), I32),
        pltpu.VMEM((4, 128), I32),
        pltpu.VMEM((4, 128), I32),
        pltpu.VMEM((512, 16), F32),
        pltpu.VMEM((512, 16), F32),
        pltpu.VMEM_SHARED((N_PAD, 16), F32),
        pltpu.SemaphoreType.DMA,
        pltpu.SemaphoreType.DMA,
        pltpu.SemaphoreType.DMA,
        pltpu.SemaphoreType.DMA,
        pltpu.SemaphoreType.DMA,
    ],
)(_agg_body)


def _scale_body(x_ref, dgo_ref, dgi_ref, xlo_ref, xhi_ref, dsi_ref):
    x = x_ref[...]
    so = lax.rsqrt(jnp.maximum(dgo_ref[0], 1.0))
    si = lax.rsqrt(jnp.maximum(dgi_ref[0], 1.0))
    xs = x * so
    xlo_ref[0] = xs[:, :16]
    xhi_ref[0] = xs[:, 16:]
    dsi_ref[0] = si


def _k_scale(x, deg6):
    return pl.pallas_call(
        _scale_body,
        grid=(3, NBLK),
        in_specs=[
            pl.BlockSpec((BLK, EMBED), lambda g, nb: (g * NBLK + nb, 0)),
            pl.BlockSpec((1, BLK, 1), lambda g, nb: (g, nb, 0)),
            pl.BlockSpec((1, BLK, 1), lambda g, nb: (3 + g, nb, 0)),
        ],
        out_specs=[
            pl.BlockSpec((1, BLK, 16), lambda g, nb: (g, nb, 0)),
            pl.BlockSpec((1, BLK, 16), lambda g, nb: (g, nb, 0)),
            pl.BlockSpec((1, BLK, 1), lambda g, nb: (g, nb, 0)),
        ],
        out_shape=[
            jax.ShapeDtypeStruct((3, N_PAD, 16), F32),
            jax.ShapeDtypeStruct((3, N_PAD, 16), F32),
            jax.ShapeDtypeStruct((3, N_PAD, 1), F32),
        ],
    )(x, deg6, deg6)


def _post_body(p00_ref, p01_ref, p10_ref, p11_ref, dsi_ref, bat_ref,
               w_ref, b_ref, out_ref):
    nb = pl.program_id(1)

    @pl.when(nb == 0)
    def _():
        out_ref[...] = jnp.zeros_like(out_ref)

    si = dsi_ref[0]
    lo = (p00_ref[0] + p01_ref[0]) * si
    hi = (p10_ref[0] + p11_ref[0]) * si
    z = jnp.concatenate([lo, hi], axis=1)
    h = jnp.maximum(jnp.dot(z, w_ref[...], preferred_element_type=F32)
                    + b_ref[...], 0.0)
    bb = bat_ref[0, :, 0]
    b0 = jnp.min(bb)
    b1 = jnp.max(bb)

    def pbody(k, acc):
        g_id = b0 + k
        m = (bb == g_id)[:, None]
        colmax = jnp.max(jnp.where(m, h, 0.0), axis=0)
        colmax = jnp.where(g_id < NG, colmax, 0.0)
        gw = jnp.minimum(g_id, NG - 1)
        oh = lax.broadcasted_iota(I32, (NG, 1), 0) == gw
        return jnp.where(oh, jnp.maximum(acc, colmax[None, :]), acc)

    acc = lax.fori_loop(0, b1 - b0 + 1, pbody, out_ref[0])
    out_ref[0] = acc


def _k_post(part, dsi3, bat3, w_pad, b_pad):
    def pspec(k):
        return pl.BlockSpec((1, BLK, 16), lambda g, nb, k=k: (g * 4 + k, nb, 0))
    return pl.pallas_call(
        _post_body,
        grid=(3, NBLK),
        in_specs=[
            pspec(0), pspec(1), pspec(2), pspec(3),
            pl.BlockSpec((1, BLK, 1), lambda g, nb: (g, nb, 0)),
            pl.BlockSpec((1, BLK, 16), lambda g, nb: (g, nb, 0)),
            pl.BlockSpec((EMBED, 128), lambda g, nb: (0, 0)),
            pl.BlockSpec((1, 128), lambda g, nb: (0, 0)),
        ],
        out_specs=pl.BlockSpec((1, NG, 128), lambda g, nb: (g, 0, 0)),
        out_shape=jax.ShapeDtypeStruct((3, NG, 128), F32),
    )(part, part, part, part, dsi3, bat3, w_pad, b_pad)


def _head_body(pool_ref, wfc_ref, bfc_ref, wdef_ref, bdef_ref,
               mmd_ref, def_ref):
    mmd = jnp.dot(pool_ref[...], wfc_ref[...],
                  preferred_element_type=F32) + bfc_ref[...]
    mmd_ref[...] = mmd
    logits = jnp.dot(mmd[0:NG], wdef_ref[...],
                     preferred_element_type=F32) + bdef_ref[...]
    def_ref[...] = jax.nn.sigmoid(logits)


def _k_head(pool2, wfc, bfc, wdef, bdef):
    return pl.pallas_call(
        _head_body,
        out_shape=[
            jax.ShapeDtypeStruct((3 * NG, FC), F32),
            jax.ShapeDtypeStruct((NG, 128), F32),
        ],
    )(pool2, wfc, bfc, wdef, bdef)


def kernel(feat, edge_index, batch, feat_src, edge_index_src, batch_src,
           feat_tar, edge_index_tar, batch_tar,
           embed, W_gcn, b_gcn, W_fc, b_fc, W_def, b_def,
           W_d1, b_d1, W_d2, b_d2):
    f32 = F32
    feats = jnp.stack([feat, feat_src, feat_tar]).astype(I32)
    feats = jnp.pad(feats, ((0, 0), (0, N_PAD - N)))
    feats2d = jnp.pad(feats.reshape(3 * N_PAD // 128, 128),
                      ((0, _EROWS - 3 * N_PAD // 128), (0, 0)))
    def pad_edges(ei):
        return jnp.pad(ei.astype(I32), ((0, 0), (0, E_ROWS * 128 - E)),
                       constant_values=TRASH).reshape(2, E_ROWS, 128)

    e0, e1, e2 = (pad_edges(edge_index), pad_edges(edge_index_src),
                  pad_edges(edge_index_tar))
    bat3 = jnp.stack([batch, batch_src, batch_tar]).astype(I32)
    bat3 = jnp.pad(bat3, ((0, 0), (0, N_PAD - N)), constant_values=NG)
    bat3 = jnp.broadcast_to(bat3[:, :, None], (3, N_PAD, 16))
    zeros1 = jnp.zeros((_SLICE,), f32)
    zeros2 = jnp.zeros((_SLICE, 16), f32)
    w_pad = jnp.pad(W_gcn.astype(f32), ((0, 0), (0, 128 - HID)))
    b_pad = jnp.pad(b_gcn.astype(f32), (0, 128 - HID)).reshape(1, 128)
    wfc = jnp.pad(W_fc.astype(f32), ((0, 128 - HID), (0, 0)))
    bfc = b_fc.astype(f32).reshape(1, FC)
    wdef = jnp.pad(W_def.astype(f32), ((0, 0), (0, 127)))
    bdef = jnp.pad(b_def.astype(f32), (0, 127)).reshape(1, 128)

    deg6 = _k_deg(e0, e1, e2, zeros1).reshape(6, N_PAD, 1)
    x = _k_emb(feats2d, embed.astype(f32))
    xs_lo3, xs_hi3, dsi3 = _k_scale(x, deg6)

    part = _k_agg(e0, e1, e2, xs_lo3, xs_hi3, zeros2)

    pooled = _k_post(part, dsi3, bat3, w_pad, b_pad)
    pool2 = pooled.reshape(3 * NG, 128)
    mmd, defect = _k_head(pool2, wfc, bfc, wdef, bdef)

    x_batch_mmd = mmd[0:NG]
    x_src_mmd = mmd[NG:2 * NG]
    x_tar_mmd = mmd[2 * NG:3 * NG]
    defect_prob = defect[:, 0:1]
    x_loss_mmd_node = jnp.zeros((), f32)
    return (x_batch_mmd, defect_prob, x_src_mmd, x_tar_mmd, x_loss_mmd_node)


# speedup vs baseline: 1.0322x; 1.0322x over previous
import functools

import jax
import jax.numpy as jnp
from jax import lax
from jax.experimental import pallas as pl
from jax.experimental.pallas import tpu as pltpu
from jax.experimental.pallas import tpu_sc as plsc

N = 100000
E = 1600000
VOCAB = 5000
EMBED = 32
HID = 100
FC = 32
NG = 64

N_PAD = 102400
E_ROWS = 12544
TRASH = N_PAD - 1
NC = 2
NS = 16
BLK = 2048
NBLK = N_PAD // BLK

F32 = jnp.float32
I32 = jnp.int32

_MESH = plsc.VectorSubcoreMesh(
    core_axis_name="c", subcore_axis_name="s", num_cores=NC, num_subcores=NS)

_SLICE = N_PAD // NS


def _deg_body(e0_ref, e1_ref, e2_ref, zeros_ref, dsw_ref, idxbA, idxbB,
              ones_v, stage_i, stage_o, hist, semA, semB):
    c = lax.axis_index("c")
    s = lax.axis_index("s")
    for b in range(8):
        ones_v[pl.ds(16 * b, 16)] = jnp.ones((16,), F32)
    rows_per_tile = E_ROWS // NS
    npairs = rows_per_tile // 16
    for g, edges_ref in enumerate((e0_ref, e1_ref, e2_ref)):
        pltpu.sync_copy(zeros_ref, hist.at[pl.ds(s * _SLICE, _SLICE)])
        plsc.subcore_barrier()

        def pair(k, carry):
            for half, (idxb, sem) in enumerate(((idxbA, semA), (idxbB, semB))):
                base = s * rows_per_tile + (2 * k + half) * 8

                @pl.when(k > 0)
                def _():
                    pltpu.make_async_copy(edges_ref.at[0, pl.ds(0, 8)],
                                          idxb, sem).wait()

                pltpu.sync_copy(edges_ref.at[c, pl.ds(base, 8)], idxb)
                for b in range(8):
                    pltpu.async_copy(ones_v, hist.at[idxb.at[b]], sem,
                                     add=True)
            return carry

        lax.fori_loop(0, npairs, pair, 0)
        for idxb, sem in ((idxbA, semA), (idxbB, semB)):
            pltpu.make_async_copy(edges_ref.at[0, pl.ds(0, 8)],
                                  idxb, sem).wait()
        plsc.subcore_barrier()

        def chunk(ci, carry):
            n0 = s * _SLICE + ci * 256
            pltpu.sync_copy(hist.at[pl.ds(n0, 256)], stage_i)
            for k in range(16):
                d = jnp.maximum(stage_i[pl.ds(16 * k, 16)], 1.0)
                i = plsc.bitcast(d, I32)
                i = 0x5F3759DF - lax.shift_right_logical(i, 1)
                y = plsc.bitcast(i, F32)
                for _ in range(3):
                    y = y * (1.5 - 0.5 * d * y * y)
                node = 16 * k + lax.iota(I32, 16)
                for f in range(16):
                    plsc.store_scatter(stage_o, [node, jnp.full((16,), f, I32)],
                                       y)
            pltpu.sync_copy(stage_o, dsw_ref.at[c * 3 + g, pl.ds(n0, 256)])
            return carry

        lax.fori_loop(0, _SLICE // 256, chunk, 0)
        plsc.subcore_barrier()


_k_deg = functools.partial(
    pl.kernel,
    out_type=jax.ShapeDtypeStruct((NC * 3, N_PAD, 16), F32),
    mesh=_MESH,
    compiler_params=pltpu.CompilerParams(use_tc_tiling_on_sc=False,
                                         needs_layout_passes=False),
    scratch_types=[
        pltpu.VMEM((8, 128), I32),
        pltpu.VMEM((8, 128), I32),
        pltpu.VMEM((128,), F32),
        pltpu.VMEM((256,), F32),
        pltpu.VMEM((256, 16), F32),
        pltpu.VMEM_SHARED((N_PAD,), F32),
        pltpu.SemaphoreType.DMA,
        pltpu.SemaphoreType.DMA,
    ],
)(_deg_body)


_EROWS = 2432


def _emb_body(feats_ref, embed_ref, x_ref, idxbA, idxbB, rowsA, rowsB,
              sem_g, semA, semB):
    wid = lax.axis_index("s") * NC + lax.axis_index("c")
    rows_per_w = _EROWS // (NC * NS)
    base0 = wid * rows_per_w

    def do_group(gi, idxb, rows, sem, first):
        base = base0 + gi * 4

        @pl.when(jnp.logical_not(first))
        def _():
            pltpu.make_async_copy(x_ref.at[pl.ds(0, 512)], rows, sem).wait()

        pltpu.sync_copy(feats_ref.at[pl.ds(base, 4)], idxb)
        cps = [pltpu.async_copy(embed_ref.at[idxb.at[b]],
                                rows.at[pl.ds(b * 128, 128)], sem_g)
               for b in range(4)]
        for cp in cps:
            cp.wait()
        for b in range(4):
            pltpu.async_copy(rows.at[pl.ds(b * 128, 128)],
                             x_ref.at[pl.ds((base + b) * 128, 128)], sem)

    def pair(k, carry):
        do_group(2 * k, idxbA, rowsA, semA, k == 0)
        do_group(2 * k + 1, idxbB, rowsB, semB, k == 0)
        return carry

    lax.fori_loop(0, 9, pair, 0)
    do_group(18, idxbA, rowsA, semA, jnp.bool_(False))
    for rows, sem in ((rowsA, semA), (rowsB, semB)):
        pltpu.make_async_copy(x_ref.at[pl.ds(0, 512)], rows, sem).wait()


_k_emb = functools.partial(
    pl.kernel,
    out_type=jax.ShapeDtypeStruct((_EROWS * 128, EMBED), F32),
    mesh=_MESH,
    compiler_params=pltpu.CompilerParams(use_tc_tiling_on_sc=False),
    scratch_types=[
        pltpu.VMEM((4, 128), I32),
        pltpu.VMEM((4, 128), I32),
        pltpu.VMEM((512, EMBED), F32),
        pltpu.VMEM((512, EMBED), F32),
        pltpu.SemaphoreType.DMA,
        pltpu.SemaphoreType.DMA,
        pltpu.SemaphoreType.DMA,
    ],
)(_emb_body)


def _agg_body(e0_ref, e1_ref, e2_ref, xlo_ref, xhi_ref,
              zeros_ref, part_ref,
              idxsA, idxdA, idxsB, idxdB, rowsA, rowsB, agg,
              sem_g, semA, semB, siA, siB):
    c = lax.axis_index("c")
    s = lax.axis_index("s")
    rows_per_sc = E_ROWS // NC
    rows_per_tile = rows_per_sc // NS
    e_tab = (e0_ref, e1_ref, e2_ref)
    for g in range(3):
        edges_ref = e_tab[g]
        for p in range(2):
            xs_ref = (xlo_ref if p == 0 else xhi_ref).at[g]
            pltpu.sync_copy(zeros_ref, agg.at[pl.ds(s * _SLICE, _SLICE)])
            plsc.subcore_barrier()
            base0 = c * rows_per_sc + s * rows_per_tile

            def fire_idx(gi, idx_s, idx_d, sem):
                base = base0 + gi * 4
                pltpu.async_copy(edges_ref.at[0, pl.ds(base, 4)], idx_s, sem)
                pltpu.async_copy(edges_ref.at[1, pl.ds(base, 4)], idx_d, sem)

            def wait_idx(idx_s, idx_d, sem):
                pltpu.make_async_copy(edges_ref.at[0, pl.ds(0, 4)],
                                      idx_s, sem).wait()
                pltpu.make_async_copy(edges_ref.at[1, pl.ds(0, 4)],
                                      idx_d, sem).wait()

            def drain_sc(rows, sem):
                pltpu.make_async_copy(zeros_ref.at[pl.ds(0, 512)],
                                      rows, sem).wait()

            def gather_scatter(idx_s, idx_d, rows, sem):
                cps = [pltpu.async_copy(xs_ref.at[idx_s.at[b]],
                                        rows.at[pl.ds(b * 128, 128)], sem_g)
                       for b in range(4)]
                for cp in cps:
                    cp.wait()
                for b in range(4):
                    pltpu.async_copy(rows.at[pl.ds(b * 128, 128)],
                                     agg.at[idx_d.at[b]], sem, add=True)

            fire_idx(0, idxsA, idxdA, siA)

            def pair(k, carry):
                @pl.when(k > 0)
                def _():
                    drain_sc(rowsB, semB)
                fire_idx(2 * k + 1, idxsB, idxdB, siB)
                wait_idx(idxsA, idxdA, siA)
                gather_scatter(idxsA, idxdA, rowsA, semA)
                wait_idx(idxsB, idxdB, siB)
                gather_scatter(idxsB, idxdB, rowsB, semB)

                @pl.when(k < 48)
                def _():
                    drain_sc(rowsA, semA)
                    fire_idx(2 * k + 2, idxsA, idxdA, siA)
                return carry

            lax.fori_loop(0, 49, pair, 0)
            drain_sc(rowsA, semA)
            drain_sc(rowsB, semB)
            plsc.subcore_barrier()
            pltpu.sync_copy(agg.at[pl.ds(s * _SLICE, _SLICE)],
                            part_ref.at[g * 4 + p * NC + c,
                                        pl.ds(s * _SLICE, _SLICE)])
            plsc.subcore_barrier()


_k_agg = functools.partial(
    pl.kernel,
    out_type=jax.ShapeDtypeStruct((3 * 2 * NC, N_PAD, 16), F32),
    mesh=_MESH,
    compiler_params=pltpu.CompilerParams(use_tc_tiling_on_sc=False),
    scratch_types=[
        pltpu.VMEM((4, 128), I32),
        pltpu.VMEM((4, 128), I32),
        pltpu.VMEM((4, 128), I32),
        pltpu.VMEM((4, 128), I32),
        pltpu.VMEM((512, 16), F32),
        pltpu.VMEM((512, 16), F32),
        pltpu.VMEM_SHARED((N_PAD, 16), F32),
        pltpu.SemaphoreType.DMA,
        pltpu.SemaphoreType.DMA,
        pltpu.SemaphoreType.DMA,
        pltpu.SemaphoreType.DMA,
        pltpu.SemaphoreType.DMA,
    ],
)(_agg_body)


def _scale_body(x_ref, dso_ref, xlo_ref, xhi_ref):
    xs = x_ref[...] * dso_ref[0][:, 0:1]
    xlo_ref[0] = xs[:, :16]
    xhi_ref[0] = xs[:, 16:]


def _k_scale(x, dsw):
    return pl.pallas_call(
        _scale_body,
        grid=(3, NBLK),
        in_specs=[
            pl.BlockSpec((BLK, EMBED), lambda g, nb: (g * NBLK + nb, 0)),
            pl.BlockSpec((1, BLK, 16), lambda g, nb: (g, nb, 0)),
        ],
        out_specs=[
            pl.BlockSpec((1, BLK, 16), lambda g, nb: (g, nb, 0)),
            pl.BlockSpec((1, BLK, 16), lambda g, nb: (g, nb, 0)),
        ],
        out_shape=[
            jax.ShapeDtypeStruct((3, N_PAD, 16), F32),
            jax.ShapeDtypeStruct((3, N_PAD, 16), F32),
        ],
    )(x, dsw)


def _post_body(p00_ref, p01_ref, p10_ref, p11_ref, dsi_ref, bat_ref,
               w_ref, b_ref, out_ref):
    nb = pl.program_id(1)

    @pl.when(nb == 0)
    def _():
        out_ref[...] = jnp.zeros_like(out_ref)

    si = dsi_ref[0][:, 0:1]
    lo = (p00_ref[0] + p01_ref[0]) * si
    hi = (p10_ref[0] + p11_ref[0]) * si
    z = jnp.concatenate([lo, hi], axis=1)
    h = jnp.maximum(jnp.dot(z, w_ref[...], preferred_element_type=F32)
                    + b_ref[...], 0.0)
    bb = bat_ref[0, :, 0]
    b0 = jnp.min(bb)
    b1 = jnp.max(bb)

    def pbody(k, acc):
        g_id = b0 + k
        m = (bb == g_id)[:, None]
        colmax = jnp.max(jnp.where(m, h, 0.0), axis=0)
        colmax = jnp.where(g_id < NG, colmax, 0.0)
        gw = jnp.minimum(g_id, NG - 1)
        oh = lax.broadcasted_iota(I32, (NG, 1), 0) == gw
        return jnp.where(oh, jnp.maximum(acc, colmax[None, :]), acc)

    acc = lax.fori_loop(0, b1 - b0 + 1, pbody, out_ref[0])
    out_ref[0] = acc


def _k_post(part, dsi3, bat3, w_pad, b_pad):
    def pspec(k):
        return pl.BlockSpec((1, BLK, 16), lambda g, nb, k=k: (g * 4 + k, nb, 0))
    return pl.pallas_call(
        _post_body,
        grid=(3, NBLK),
        in_specs=[
            pspec(0), pspec(1), pspec(2), pspec(3),
            pl.BlockSpec((1, BLK, 16), lambda g, nb: (3 + g, nb, 0)),
            pl.BlockSpec((1, BLK, 1), lambda g, nb: (g, nb, 0)),
            pl.BlockSpec((EMBED, 128), lambda g, nb: (0, 0)),
            pl.BlockSpec((1, 128), lambda g, nb: (0, 0)),
        ],
        out_specs=pl.BlockSpec((1, NG, 128), lambda g, nb: (g, 0, 0)),
        out_shape=jax.ShapeDtypeStruct((3, NG, 128), F32),
    )(part, part, part, part, dsi3, bat3, w_pad, b_pad)


def _head_body(pool_ref, wfc_ref, bfc_ref, wdef_ref, bdef_ref,
               mmd_ref, def_ref):
    mmd = jnp.dot(pool_ref[...], wfc_ref[...],
                  preferred_element_type=F32) + bfc_ref[...]
    mmd_ref[...] = mmd
    logits = jnp.dot(mmd[0:NG], wdef_ref[...],
                     preferred_element_type=F32) + bdef_ref[...]
    def_ref[...] = jax.nn.sigmoid(logits)


def _k_head(pool2, wfc, bfc, wdef, bdef):
    return pl.pallas_call(
        _head_body,
        out_shape=[
            jax.ShapeDtypeStruct((3 * NG, FC), F32),
            jax.ShapeDtypeStruct((NG, 128), F32),
        ],
    )(pool2, wfc, bfc, wdef, bdef)


def kernel(feat, edge_index, batch, feat_src, edge_index_src, batch_src,
           feat_tar, edge_index_tar, batch_tar,
           embed, W_gcn, b_gcn, W_fc, b_fc, W_def, b_def,
           W_d1, b_d1, W_d2, b_d2):
    f32 = F32
    feats = jnp.stack([feat, feat_src, feat_tar]).astype(I32)
    feats = jnp.pad(feats, ((0, 0), (0, N_PAD - N)))
    feats2d = jnp.pad(feats.reshape(3 * N_PAD // 128, 128),
                      ((0, _EROWS - 3 * N_PAD // 128), (0, 0)))
    def pad_edges(ei):
        return jnp.pad(ei.astype(I32), ((0, 0), (0, E_ROWS * 128 - E)),
                       constant_values=TRASH).reshape(2, E_ROWS, 128)

    e0, e1, e2 = (pad_edges(edge_index), pad_edges(edge_index_src),
                  pad_edges(edge_index_tar))
    bat3 = jnp.stack([batch, batch_src, batch_tar]).astype(I32)
    bat3 = jnp.pad(bat3, ((0, 0), (0, N_PAD - N)),
                   constant_values=NG).reshape(3, N_PAD, 1)
    zeros1 = jnp.zeros((_SLICE,), f32)
    zeros2 = jnp.zeros((_SLICE, 16), f32)
    w_pad = jnp.pad(W_gcn.astype(f32), ((0, 0), (0, 128 - HID)))
    b_pad = jnp.pad(b_gcn.astype(f32), (0, 128 - HID)).reshape(1, 128)
    wfc = jnp.pad(W_fc.astype(f32), ((0, 128 - HID), (0, 0)))
    bfc = b_fc.astype(f32).reshape(1, FC)
    wdef = jnp.pad(W_def.astype(f32), ((0, 0), (0, 127)))
    bdef = jnp.pad(b_def.astype(f32), (0, 127)).reshape(1, 128)

    dsw = _k_deg(e0, e1, e2, zeros1)
    x = _k_emb(feats2d, embed.astype(f32))
    xs_lo3, xs_hi3 = _k_scale(x, dsw)

    part = _k_agg(e0, e1, e2, xs_lo3, xs_hi3, zeros2)

    pooled = _k_post(part, dsw, bat3, w_pad, b_pad)
    pool2 = pooled.reshape(3 * NG, 128)
    mmd, defect = _k_head(pool2, wfc, bfc, wdef, bdef)

    x_batch_mmd = mmd[0:NG]
    x_src_mmd = mmd[NG:2 * NG]
    x_tar_mmd = mmd[2 * NG:3 * NG]
    defect_prob = defect[:, 0:1]
    x_loss_mmd_node = jnp.zeros((), f32)
    return (x_batch_mmd, defect_prob, x_src_mmd, x_tar_mmd, x_loss_mmd_node)


# speedup vs baseline: 1.1909x; 1.1538x over previous
import functools

import jax
import jax.numpy as jnp
from jax import lax
from jax.experimental import pallas as pl
from jax.experimental.pallas import tpu as pltpu
from jax.experimental.pallas import tpu_sc as plsc

N = 100000
E = 1600000
VOCAB = 5000
EMBED = 32
HID = 100
FC = 32
NG = 64

N_PAD = 102400
E_ROWS = 12544
TRASH = N_PAD - 1
NC = 2
NS = 16
BLK = 2048
NBLK = N_PAD // BLK

F32 = jnp.float32
I32 = jnp.int32

_MESH = plsc.VectorSubcoreMesh(
    core_axis_name="c", subcore_axis_name="s", num_cores=NC, num_subcores=NS)

_SLICE = N_PAD // NS


def _deg_body(e0_ref, e1_ref, e2_ref, zeros_ref, dsw_ref, idxbA, idxbB,
              ones_v, stage_i, stage_o, hist, semA, semB):
    c = lax.axis_index("c")
    s = lax.axis_index("s")
    for b in range(8):
        ones_v[pl.ds(16 * b, 16)] = jnp.ones((16,), F32)
    rows_per_tile = E_ROWS // NS
    npairs = rows_per_tile // 16
    for g, edges_ref in enumerate((e0_ref, e1_ref, e2_ref)):
        pltpu.sync_copy(zeros_ref, hist.at[pl.ds(s * _SLICE, _SLICE)])
        plsc.subcore_barrier()

        def pair(k, carry):
            for half, (idxb, sem) in enumerate(((idxbA, semA), (idxbB, semB))):
                base = s * rows_per_tile + (2 * k + half) * 8

                @pl.when(k > 0)
                def _():
                    pltpu.make_async_copy(edges_ref.at[0, pl.ds(0, 8)],
                                          idxb, sem).wait()

                pltpu.sync_copy(edges_ref.at[c, pl.ds(base, 8)], idxb)
                for b in range(8):
                    pltpu.async_copy(ones_v, hist.at[idxb.at[b]], sem,
                                     add=True)
            return carry

        lax.fori_loop(0, npairs, pair, 0)
        for idxb, sem in ((idxbA, semA), (idxbB, semB)):
            pltpu.make_async_copy(edges_ref.at[0, pl.ds(0, 8)],
                                  idxb, sem).wait()
        plsc.subcore_barrier()

        def chunk(ci, carry):
            n0 = s * _SLICE + ci * 256
            pltpu.sync_copy(hist.at[pl.ds(n0, 256)], stage_i)
            for k in range(16):
                d = jnp.maximum(stage_i[pl.ds(16 * k, 16)], 1.0)
                i = plsc.bitcast(d, I32)
                i = 0x5F3759DF - lax.shift_right_logical(i, 1)
                y = plsc.bitcast(i, F32)
                for _ in range(3):
                    y = y * (1.5 - 0.5 * d * y * y)
                node = 16 * k + lax.iota(I32, 16)
                for f in range(16):
                    plsc.store_scatter(stage_o, [node, jnp.full((16,), f, I32)],
                                       y)
            pltpu.sync_copy(stage_o,
                            dsw_ref.at[pl.ds((c * 3 + g) * N_PAD + n0, 256)])
            return carry

        lax.fori_loop(0, _SLICE // 256, chunk, 0)
        plsc.subcore_barrier()


_k_deg = functools.partial(
    pl.kernel,
    out_type=jax.ShapeDtypeStruct((NC * 3 * N_PAD, 16), F32),
    mesh=_MESH,
    compiler_params=pltpu.CompilerParams(use_tc_tiling_on_sc=False,
                                         needs_layout_passes=False),
    scratch_types=[
        pltpu.VMEM((8, 128), I32),
        pltpu.VMEM((8, 128), I32),
        pltpu.VMEM((128,), F32),
        pltpu.VMEM((256,), F32),
        pltpu.VMEM((256, 16), F32),
        pltpu.VMEM_SHARED((N_PAD,), F32),
        pltpu.SemaphoreType.DMA,
        pltpu.SemaphoreType.DMA,
    ],
)(_deg_body)


_EROWS = 2400


def _emb_body(feats_ref, embed_ref, dsw_ref, xlo_ref, xhi_ref,
              idxbA, idxbB, rowsA, rowsB, dstA, dstB, oloA, oloB, ohiA, ohiB,
              sem_g, semA, semB):
    wid = lax.axis_index("s") * NC + lax.axis_index("c")
    base0 = wid * (_EROWS // (NC * NS))

    def do_group(gi, nrows, idxb, rows, dst, olo, ohi, sem, first):
        base = base0 + gi * 4
        node0 = base * 128
        nn = nrows * 128

        @pl.when(jnp.logical_not(first))
        def _():
            pltpu.make_async_copy(xlo_ref.at[pl.ds(0, 512)], olo, sem).wait()
            pltpu.make_async_copy(xlo_ref.at[pl.ds(0, 512)], ohi, sem).wait()

        pltpu.sync_copy(feats_ref.at[pl.ds(base, nrows)],
                        idxb.at[pl.ds(0, nrows)])
        cps = [pltpu.async_copy(embed_ref.at[idxb.at[b]],
                                rows.at[pl.ds(b * 128, 128)], sem_g)
               for b in range(nrows)]
        pltpu.sync_copy(dsw_ref.at[pl.ds(node0, nn)], dst.at[pl.ds(0, nn)])
        for cp in cps:
            cp.wait()

        def kbody(k, carry):
            node = 16 * k + lax.iota(I32, 16)
            so = plsc.load_gather(dst, [node, jnp.zeros((16,), I32)])
            for f in range(EMBED):
                col = plsc.load_gather(rows, [node, jnp.full((16,), f, I32)])
                v = col * so
                tgt = olo if f < 16 else ohi
                plsc.store_scatter(tgt, [node, jnp.full((16,), f % 16, I32)],
                                   v)
            return carry

        lax.fori_loop(0, nn // 16, kbody, 0)
        pltpu.async_copy(olo.at[pl.ds(0, nn)], xlo_ref.at[pl.ds(node0, nn)],
                         sem)
        pltpu.async_copy(ohi.at[pl.ds(0, nn)], xhi_ref.at[pl.ds(node0, nn)],
                         sem)

    def pair(k, carry):
        do_group(2 * k, 4, idxbA, rowsA, dstA, oloA, ohiA, semA, k == 0)
        do_group(2 * k + 1, 4, idxbB, rowsB, dstB, oloB, ohiB, semB, k == 0)
        return carry

    lax.fori_loop(0, 9, pair, 0)
    do_group(18, 3, idxbA, rowsA, dstA, oloA, ohiA, semA, jnp.bool_(False))
    pltpu.make_async_copy(xlo_ref.at[pl.ds(0, 384)], oloA.at[pl.ds(0, 384)],
                          semA).wait()
    pltpu.make_async_copy(xlo_ref.at[pl.ds(0, 384)], ohiA.at[pl.ds(0, 384)],
                          semA).wait()
    pltpu.make_async_copy(xlo_ref.at[pl.ds(0, 512)], oloB, semB).wait()
    pltpu.make_async_copy(xlo_ref.at[pl.ds(0, 512)], ohiB, semB).wait()


_k_emb = functools.partial(
    pl.kernel,
    out_type=[jax.ShapeDtypeStruct((3 * N_PAD, 16), F32),
              jax.ShapeDtypeStruct((3 * N_PAD, 16), F32)],
    mesh=_MESH,
    compiler_params=pltpu.CompilerParams(use_tc_tiling_on_sc=False,
                                         needs_layout_passes=False),
    scratch_types=[
        pltpu.VMEM((4, 128), I32),
        pltpu.VMEM((4, 128), I32),
        pltpu.VMEM((512, EMBED), F32),
        pltpu.VMEM((512, EMBED), F32),
        pltpu.VMEM((512, 16), F32),
        pltpu.VMEM((512, 16), F32),
        pltpu.VMEM((512, 16), F32),
        pltpu.VMEM((512, 16), F32),
        pltpu.VMEM((512, 16), F32),
        pltpu.VMEM((512, 16), F32),
        pltpu.SemaphoreType.DMA,
        pltpu.SemaphoreType.DMA,
        pltpu.SemaphoreType.DMA,
    ],
)(_emb_body)


def _agg_body(e0_ref, e1_ref, e2_ref, xlo_ref, xhi_ref,
              zeros_ref, part_ref,
              idxsA, idxdA, idxsB, idxdB, rowsA, rowsB, agg,
              sem_g, semA, semB, siA, siB):
    c = lax.axis_index("c")
    s = lax.axis_index("s")
    rows_per_sc = E_ROWS // NC
    rows_per_tile = rows_per_sc // NS
    e_tab = (e0_ref, e1_ref, e2_ref)
    for g in range(3):
        edges_ref = e_tab[g]
        for p in range(2):
            xs_ref = (xlo_ref if p == 0 else
                      xhi_ref).at[pl.ds(g * N_PAD, N_PAD)]
            pltpu.sync_copy(zeros_ref, agg.at[pl.ds(s * _SLICE, _SLICE)])
            plsc.subcore_barrier()
            base0 = c * rows_per_sc + s * rows_per_tile

            def fire_idx(gi, idx_s, idx_d, sem):
                base = base0 + gi * 4
                pltpu.async_copy(edges_ref.at[0, pl.ds(base, 4)], idx_s, sem)
                pltpu.async_copy(edges_ref.at[1, pl.ds(base, 4)], idx_d, sem)

            def wait_idx(idx_s, idx_d, sem):
                pltpu.make_async_copy(edges_ref.at[0, pl.ds(0, 4)],
                                      idx_s, sem).wait()
                pltpu.make_async_copy(edges_ref.at[1, pl.ds(0, 4)],
                                      idx_d, sem).wait()

            def drain_sc(rows, sem):
                pltpu.make_async_copy(zeros_ref.at[pl.ds(0, 512)],
                                      rows, sem).wait()

            def gather_scatter(idx_s, idx_d, rows, sem):
                cps = [pltpu.async_copy(xs_ref.at[idx_s.at[b]],
                                        rows.at[pl.ds(b * 128, 128)], sem_g)
                       for b in range(4)]
                for cp in cps:
                    cp.wait()
                for b in range(4):
                    pltpu.async_copy(rows.at[pl.ds(b * 128, 128)],
                                     agg.at[idx_d.at[b]], sem, add=True)

            fire_idx(0, idxsA, idxdA, siA)

            def pair(k, carry):
                @pl.when(k > 0)
                def _():
                    drain_sc(rowsB, semB)
                fire_idx(2 * k + 1, idxsB, idxdB, siB)
                wait_idx(idxsA, idxdA, siA)
                gather_scatter(idxsA, idxdA, rowsA, semA)
                wait_idx(idxsB, idxdB, siB)
                gather_scatter(idxsB, idxdB, rowsB, semB)

                @pl.when(k < 48)
                def _():
                    drain_sc(rowsA, semA)
                    fire_idx(2 * k + 2, idxsA, idxdA, siA)
                return carry

            lax.fori_loop(0, 49, pair, 0)
            drain_sc(rowsA, semA)
            drain_sc(rowsB, semB)
            plsc.subcore_barrier()
            pltpu.sync_copy(agg.at[pl.ds(s * _SLICE, _SLICE)],
                            part_ref.at[g * 4 + p * NC + c,
                                        pl.ds(s * _SLICE, _SLICE)])
            plsc.subcore_barrier()


_k_agg = functools.partial(
    pl.kernel,
    out_type=jax.ShapeDtypeStruct((3 * 2 * NC, N_PAD, 16), F32),
    mesh=_MESH,
    compiler_params=pltpu.CompilerParams(use_tc_tiling_on_sc=False),
    scratch_types=[
        pltpu.VMEM((4, 128), I32),
        pltpu.VMEM((4, 128), I32),
        pltpu.VMEM((4, 128), I32),
        pltpu.VMEM((4, 128), I32),
        pltpu.VMEM((512, 16), F32),
        pltpu.VMEM((512, 16), F32),
        pltpu.VMEM_SHARED((N_PAD, 16), F32),
        pltpu.SemaphoreType.DMA,
        pltpu.SemaphoreType.DMA,
        pltpu.SemaphoreType.DMA,
        pltpu.SemaphoreType.DMA,
        pltpu.SemaphoreType.DMA,
    ],
)(_agg_body)


def _post_body(p00_ref, p01_ref, p10_ref, p11_ref, dsi_ref, bat_ref,
               w_ref, b_ref, out_ref):
    nb = pl.program_id(1)

    @pl.when(nb == 0)
    def _():
        out_ref[...] = jnp.zeros_like(out_ref)

    si = dsi_ref[:, 0:1]
    lo = (p00_ref[0] + p01_ref[0]) * si
    hi = (p10_ref[0] + p11_ref[0]) * si
    z = jnp.concatenate([lo, hi], axis=1)
    h = jnp.maximum(jnp.dot(z, w_ref[...], preferred_element_type=F32)
                    + b_ref[...], 0.0)
    bb = bat_ref[0, :, 0]
    b0 = jnp.min(bb)
    b1 = jnp.max(bb)

    def pbody(k, acc):
        g_id = b0 + k
        m = (bb == g_id)[:, None]
        colmax = jnp.max(jnp.where(m, h, 0.0), axis=0)
        colmax = jnp.where(g_id < NG, colmax, 0.0)
        gw = jnp.minimum(g_id, NG - 1)
        oh = lax.broadcasted_iota(I32, (NG, 1), 0) == gw
        return jnp.where(oh, jnp.maximum(acc, colmax[None, :]), acc)

    acc = lax.fori_loop(0, b1 - b0 + 1, pbody, out_ref[0])
    out_ref[0] = acc


def _k_post(part, dsi3, bat3, w_pad, b_pad):
    def pspec(k):
        return pl.BlockSpec((1, BLK, 16), lambda g, nb, k=k: (g * 4 + k, nb, 0))
    return pl.pallas_call(
        _post_body,
        grid=(3, NBLK),
        in_specs=[
            pspec(0), pspec(1), pspec(2), pspec(3),
            pl.BlockSpec((BLK, 16), lambda g, nb: ((3 + g) * NBLK + nb, 0)),
            pl.BlockSpec((1, BLK, 1), lambda g, nb: (g, nb, 0)),
            pl.BlockSpec((EMBED, 128), lambda g, nb: (0, 0)),
            pl.BlockSpec((1, 128), lambda g, nb: (0, 0)),
        ],
        out_specs=pl.BlockSpec((1, NG, 128), lambda g, nb: (g, 0, 0)),
        out_shape=jax.ShapeDtypeStruct((3, NG, 128), F32),
    )(part, part, part, part, dsi3, bat3, w_pad, b_pad)


def _head_body(pool_ref, wfc_ref, bfc_ref, wdef_ref, bdef_ref,
               mmd_ref, def_ref):
    mmd = jnp.dot(pool_ref[...], wfc_ref[...],
                  preferred_element_type=F32) + bfc_ref[...]
    mmd_ref[...] = mmd
    logits = jnp.dot(mmd[0:NG], wdef_ref[...],
                     preferred_element_type=F32) + bdef_ref[...]
    def_ref[...] = jax.nn.sigmoid(logits)


def _k_head(pool2, wfc, bfc, wdef, bdef):
    return pl.pallas_call(
        _head_body,
        out_shape=[
            jax.ShapeDtypeStruct((3 * NG, FC), F32),
            jax.ShapeDtypeStruct((NG, 128), F32),
        ],
    )(pool2, wfc, bfc, wdef, bdef)


def kernel(feat, edge_index, batch, feat_src, edge_index_src, batch_src,
           feat_tar, edge_index_tar, batch_tar,
           embed, W_gcn, b_gcn, W_fc, b_fc, W_def, b_def,
           W_d1, b_d1, W_d2, b_d2):
    f32 = F32
    feats = jnp.stack([feat, feat_src, feat_tar]).astype(I32)
    feats = jnp.pad(feats, ((0, 0), (0, N_PAD - N)))
    feats2d = feats.reshape(_EROWS, 128)
    def pad_edges(ei):
        return jnp.pad(ei.astype(I32), ((0, 0), (0, E_ROWS * 128 - E)),
                       constant_values=TRASH).reshape(2, E_ROWS, 128)

    e0, e1, e2 = (pad_edges(edge_index), pad_edges(edge_index_src),
                  pad_edges(edge_index_tar))
    bat3 = jnp.stack([batch, batch_src, batch_tar]).astype(I32)
    bat3 = jnp.pad(bat3, ((0, 0), (0, N_PAD - N)),
                   constant_values=NG).reshape(3, N_PAD, 1)
    zeros1 = jnp.zeros((_SLICE,), f32)
    zeros2 = jnp.zeros((_SLICE, 16), f32)
    w_pad = jnp.pad(W_gcn.astype(f32), ((0, 0), (0, 128 - HID)))
    b_pad = jnp.pad(b_gcn.astype(f32), (0, 128 - HID)).reshape(1, 128)
    wfc = jnp.pad(W_fc.astype(f32), ((0, 128 - HID), (0, 0)))
    bfc = b_fc.astype(f32).reshape(1, FC)
    wdef = jnp.pad(W_def.astype(f32), ((0, 0), (0, 127)))
    bdef = jnp.pad(b_def.astype(f32), (0, 127)).reshape(1, 128)

    dsw = _k_deg(e0, e1, e2, zeros1)
    xs_lo3, xs_hi3 = _k_emb(feats2d, embed.astype(f32), dsw)

    part = _k_agg(e0, e1, e2, xs_lo3, xs_hi3, zeros2)

    pooled = _k_post(part, dsw, bat3, w_pad, b_pad)
    pool2 = pooled.reshape(3 * NG, 128)
    mmd, defect = _k_head(pool2, wfc, bfc, wdef, bdef)

    x_batch_mmd = mmd[0:NG]
    x_src_mmd = mmd[NG:2 * NG]
    x_tar_mmd = mmd[2 * NG:3 * NG]
    defect_prob = defect[:, 0:1]
    x_loss_mmd_node = jnp.zeros((), f32)
    return (x_batch_mmd, defect_prob, x_src_mmd, x_tar_mmd, x_loss_mmd_node)
